# baseline (speedup 1.0000x reference)
import functools

import jax
import jax.numpy as jnp
from jax.experimental import pallas as pl
from jax.experimental.pallas import tpu as pltpu

N, C, S = 50000, 128, 8
BN = 5000


def _route_kernel(f_ref, w_ref, b_ref, out_ref):
    f = f_ref[...]
    logits = jnp.dot(f, w_ref[...], preferred_element_type=jnp.float32)
    logits = logits + b_ref[...]
    idx = jnp.argmax(logits, axis=1)
    sel = idx[None, :, None] == jax.lax.broadcasted_iota(jnp.int32, (S, BN, 1), 0)
    out_ref[...] = jnp.where(sel, f[None, :, :], 0.0)


def _part(features, W, b, n_rows):
    return pl.pallas_call(
        _route_kernel,
        grid=(n_rows // BN,),
        in_specs=[
            pl.BlockSpec((BN, C), lambda i: (i, 0)),
            pl.BlockSpec((C, S), lambda i: (0, 0)),
            pl.BlockSpec((S,), lambda i: (0,)),
        ],
        out_specs=pl.BlockSpec((S, BN, C), lambda i: (0, i, 0)),
        out_shape=jax.ShapeDtypeStruct((S, n_rows, C), jnp.float32),
        compiler_params=pltpu.CompilerParams(
            dimension_semantics=("parallel",),
        ),
    )(features, W, b)


@functools.partial(jax.jit, static_argnames=())
def kernel(features, W, b):
    n1 = 25000
    a = _part(features[:n1], W, b, n1)
    bb = _part(features[n1:], W, b, N - n1)
    return jnp.concatenate([a, bb], axis=1)
